# Initial kernel scaffold; baseline (speedup 1.0000x reference)
import jax
import jax.numpy as jnp
from jax.experimental import pallas as pl


def kernel(x_in, edge_index_in, batch_in, x_out, edge_index_out, batch_out, W1a, W1b, W2a, W2b):
    raise NotImplementedError("write your pallas kernel here")


# speedup vs baseline: 180.6993x; 180.6993x over previous
import functools

import jax
import jax.numpy as jnp
from jax import lax
from jax.experimental import pallas as pl
from jax.experimental.pallas import tpu as pltpu
from jax.experimental.pallas import tpu_sc as plsc

_G = 1024
_N1, _E1 = 100000, 6400000
_N2, _E2 = 50000, 1600000
_P1 = 100352
_P2 = 50176
_NTILES = 32
_ECHUNK = 2000


def _prep_body(x_ref, b_ref, w1a_ref, w1bt_ref, w2a_ref, w2bt_ref,
               packed_ref, z_ref):
    a1 = w1a_ref[...]
    p1 = a1 * w1bt_ref[...]
    cp1 = jnp.sum(jnp.where(a1 > 0, p1, 0.0))
    cn1 = jnp.sum(jnp.where(a1 < 0, p1, 0.0))
    a2 = w2a_ref[...]
    p2 = a2 * w2bt_ref[...]
    cp2 = jnp.sum(jnp.where(a2 > 0, p2, 0.0))
    cn2 = jnp.sum(jnp.where(a2 < 0, p2, 0.0))
    x = x_ref[...]
    xp = jnp.maximum(x, 0.0)
    xn = jnp.minimum(x, 0.0)
    y = cp1 * xp + cn1 * xn
    z_ref[...] = cp2 * xp + cn2 * xn
    u = lax.bitcast_convert_type(y, jnp.int32)
    u = (u + 0x800) & jnp.int32(-4096)
    packed_ref[...] = u | b_ref[...]


def _prep(x2d, b2d, w1a, w1bt, w2a, w2bt):
    rows = x2d.shape[0]
    return pl.pallas_call(
        _prep_body,
        out_shape=(
            jax.ShapeDtypeStruct((rows, 128), jnp.int32),
            jax.ShapeDtypeStruct((rows, 128), jnp.float32),
        ),
    )(x2d, b2d, w1a, w1bt, w2a, w2bt)


def _reduce_body(part_ref, out_ref):
    out_ref[...] = jnp.sum(part_ref[...], axis=0, keepdims=True)


def _reduce(part):
    return pl.pallas_call(
        _reduce_body,
        out_shape=jax.ShapeDtypeStruct((1, 2 * _G), jnp.float32),
    )(part)


@functools.partial(
    pl.kernel,
    out_type=jax.ShapeDtypeStruct((_NTILES, 2 * _G), jnp.float32),
    mesh=plsc.VectorSubcoreMesh(core_axis_name="c", subcore_axis_name="s"),
    compiler_params=pltpu.CompilerParams(needs_layout_passes=False),
    scratch_types=[
        pltpu.VMEM((_P1,), jnp.int32),
        pltpu.VMEM((_ECHUNK,), jnp.int32),
        pltpu.VMEM((_ECHUNK,), jnp.int32),
        pltpu.VMEM((_P1 // _NTILES,), jnp.float32),
        pltpu.VMEM((_P1 // _NTILES,), jnp.int32),
        pltpu.VMEM((2 * _G,), jnp.float32),
    ],
)
def _sc_main(packed_in, packed_out, ei_in, ei_out, z_in, b_in, z_out, b_out,
             part_hbm, table, sbuf, dbuf, zbuf, bbuf, bins):
    wid = lax.axis_index("s") * 2 + lax.axis_index("c")
    hi_mask = jnp.full((16,), -4096, jnp.int32)
    lo_mask = jnp.full((16,), 4095, jnp.int32)

    def zero_bins(i, _):
        bins[pl.ds(i * 16, 16)] = jnp.zeros((16,), jnp.float32)
        return 0

    lax.fori_loop(0, (2 * _G) // 16, zero_bins, 0)

    def node_pass(z_hbm, b_hbm, nn, off):
        pltpu.sync_copy(z_hbm.at[pl.ds(wid * nn, nn)], zbuf.at[pl.ds(0, nn)])
        pltpu.sync_copy(b_hbm.at[pl.ds(wid * nn, nn)], bbuf.at[pl.ds(0, nn)])

        def body(j, _):
            zv = zbuf[pl.ds(j * 16, 16)]
            bv = bbuf[pl.ds(j * 16, 16)] + off
            plsc.addupdate_scatter(bins, [bv], zv)
            return 0

        lax.fori_loop(0, nn // 16, body, 0)

    def edge_pass(ei_hbm, per_tile, nedges, off):
        sbase = wid * per_tile
        dbase = nedges + wid * per_tile
        offv = jnp.full((16,), off, jnp.int32)

        def chunk(k, _):
            pltpu.sync_copy(ei_hbm.at[pl.ds(sbase + k * _ECHUNK, _ECHUNK)], sbuf)
            pltpu.sync_copy(ei_hbm.at[pl.ds(dbase + k * _ECHUNK, _ECHUNK)], dbuf)

            def body(j, _):
                vs = sbuf[pl.ds(j * 16, 16)]
                vd = dbuf[pl.ds(j * 16, 16)]
                gs = plsc.load_gather(table, [vs])
                gd = plsc.load_gather(table, [vd])
                y = plsc.bitcast(gs & hi_mask, jnp.float32)
                bd = (gd & lo_mask) + offv
                plsc.addupdate_scatter(bins, [bd], y)
                return 0

            lax.fori_loop(0, _ECHUNK // 16, body, 0)
            return 0

        lax.fori_loop(0, per_tile // _ECHUNK, chunk, 0)

    pltpu.sync_copy(packed_in, table)
    node_pass(z_in, b_in, _P1 // _NTILES, 0)
    edge_pass(ei_in, _E1 // _NTILES, _E1, 0)

    pltpu.sync_copy(packed_out, table.at[pl.ds(0, _P2)])
    node_pass(z_out, b_out, _P2 // _NTILES, _G)
    edge_pass(ei_out, _E2 // _NTILES, _E2, _G)

    pltpu.sync_copy(bins, part_hbm.at[wid])


def kernel(x_in, edge_index_in, batch_in, x_out, edge_index_out, batch_out,
           W1a, W1b, W2a, W2b):
    xi = jnp.pad(x_in[:, 0], (0, _P1 - _N1)).reshape(-1, 128)
    bi = jnp.pad(batch_in, (0, _P1 - _N1))
    xo = jnp.pad(x_out[:, 0], (0, _P2 - _N2)).reshape(-1, 128)
    bo = jnp.pad(batch_out, (0, _P2 - _N2))
    w1bt = W1b.reshape(1, 128)
    w2bt = W2b.reshape(1, 128)

    packed_in, z_in = _prep(xi, bi.reshape(-1, 128), W1a, w1bt, W2a, w2bt)
    packed_out, z_out = _prep(xo, bo.reshape(-1, 128), W1a, w1bt, W2a, w2bt)

    part = _sc_main(
        packed_in.reshape(-1), packed_out.reshape(-1),
        edge_index_in.reshape(-1), edge_index_out.reshape(-1),
        z_in.reshape(-1), bi, z_out.reshape(-1), bo,
    )
    total = _reduce(part)[0]
    return (total[:_G].reshape(_G, 1), total[_G:].reshape(_G, 1))


# speedup vs baseline: 256.5711x; 1.4199x over previous
import functools

import jax
import jax.numpy as jnp
from jax import lax
from jax.experimental import pallas as pl
from jax.experimental.pallas import tpu as pltpu
from jax.experimental.pallas import tpu_sc as plsc

_G = 1024
_N1, _E1 = 100000, 6400000
_N2, _E2 = 50000, 1600000
_P1 = 100352
_P2 = 50176
_NTILES = 32
_ECHUNK = 2000


def _prep_body(x_ref, b_ref, w1a_ref, w1bt_ref, w2a_ref, w2bt_ref,
               packed_ref, z_ref):
    a1 = w1a_ref[...]
    p1 = a1 * w1bt_ref[...]
    cp1 = jnp.sum(jnp.where(a1 > 0, p1, 0.0))
    cn1 = jnp.sum(jnp.where(a1 < 0, p1, 0.0))
    a2 = w2a_ref[...]
    p2 = a2 * w2bt_ref[...]
    cp2 = jnp.sum(jnp.where(a2 > 0, p2, 0.0))
    cn2 = jnp.sum(jnp.where(a2 < 0, p2, 0.0))
    x = x_ref[...]
    xp = jnp.maximum(x, 0.0)
    xn = jnp.minimum(x, 0.0)
    y = cp1 * xp + cn1 * xn
    z_ref[...] = cp2 * xp + cn2 * xn
    u = lax.bitcast_convert_type(y, jnp.int32)
    u = (u + 0x800) & jnp.int32(-4096)
    packed_ref[...] = u | b_ref[...]


def _prep(x2d, b2d, w1a, w1bt, w2a, w2bt):
    rows = x2d.shape[0]
    return pl.pallas_call(
        _prep_body,
        out_shape=(
            jax.ShapeDtypeStruct((rows, 128), jnp.int32),
            jax.ShapeDtypeStruct((rows, 128), jnp.float32),
        ),
    )(x2d, b2d, w1a, w1bt, w2a, w2bt)


def _reduce_body(part_ref, out_ref):
    out_ref[...] = jnp.sum(part_ref[...], axis=0, keepdims=True)


def _reduce(part):
    return pl.pallas_call(
        _reduce_body,
        out_shape=jax.ShapeDtypeStruct((1, 2 * _G), jnp.float32),
    )(part)


@functools.partial(
    pl.kernel,
    out_type=jax.ShapeDtypeStruct((_NTILES, 2 * _G), jnp.float32),
    mesh=plsc.VectorSubcoreMesh(core_axis_name="c", subcore_axis_name="s"),
    compiler_params=pltpu.CompilerParams(needs_layout_passes=False),
    scratch_types=[
        pltpu.VMEM((_N1,), jnp.int32),
        pltpu.VMEM((_ECHUNK,), jnp.int32),
        pltpu.VMEM((_ECHUNK,), jnp.int32),
        pltpu.VMEM((_P1 // _NTILES,), jnp.float32),
        pltpu.VMEM((_P1 // _NTILES,), jnp.int32),
        pltpu.VMEM((16 * _G,), jnp.float32),
        pltpu.SemaphoreType.DMA,
    ],
)
def _sc_main(packed_in, packed_out, ei_in, ei_out, z_in, b_in, z_out, b_out,
             part_hbm, table, sbuf, dbuf, zbuf, bbuf, bins, tsem):
    wid = lax.axis_index("s") * 2 + lax.axis_index("c")
    hi_mask = jnp.full((16,), -4096, jnp.int32)
    lo_mask = jnp.full((16,), 4095, jnp.int32)
    lane_off = lax.iota(jnp.int32, 16) * _G

    def zero_bins():
        def body(i, _):
            bins[pl.ds(i * 16, 16)] = jnp.zeros((16,), jnp.float32)
            return 0

        lax.fori_loop(0, (16 * _G) // 16, body, 0)

    def node_pass(z_hbm, b_hbm, nn):
        pltpu.sync_copy(z_hbm.at[pl.ds(wid * nn, nn)], zbuf.at[pl.ds(0, nn)])
        pltpu.sync_copy(b_hbm.at[pl.ds(wid * nn, nn)], bbuf.at[pl.ds(0, nn)])

        def body(j, _):
            zv = zbuf[pl.ds(j * 16, 16)]
            bv = bbuf[pl.ds(j * 16, 16)] + lane_off
            plsc.addupdate_scatter(bins, [bv], zv)
            return 0

        lax.fori_loop(0, nn // 16, body, 0)

    def edge_pass(ei_hbm, per_tile, nedges):
        sbase = wid * per_tile
        dbase = nedges + wid * per_tile

        def chunk(k, _):
            pltpu.sync_copy(ei_hbm.at[pl.ds(sbase + k * _ECHUNK, _ECHUNK)], sbuf)
            pltpu.sync_copy(ei_hbm.at[pl.ds(dbase + k * _ECHUNK, _ECHUNK)], dbuf)

            @plsc.parallel_loop(0, _ECHUNK // 16, unroll=5)
            def body(j):
                vs = sbuf[pl.ds(j * 16, 16)]
                vd = dbuf[pl.ds(j * 16, 16)]
                gs = plsc.load_gather(table, [vs])
                gd = plsc.load_gather(table, [vd])
                y = plsc.bitcast(gs & hi_mask, jnp.float32)
                bd = (gd & lo_mask) + lane_off
                plsc.addupdate_scatter(bins, [bd], y)

            return 0

        lax.fori_loop(0, per_tile // _ECHUNK, chunk, 0)

    def fold_and_dump(col):
        def fold(j, _):
            v = bins[pl.ds(j * 16, 16)]
            for l in range(1, 16):
                v = v + bins[pl.ds(l * _G + j * 16, 16)]
            bins[pl.ds(j * 16, 16)] = v
            return 0

        lax.fori_loop(0, _G // 16, fold, 0)
        pltpu.sync_copy(bins.at[pl.ds(0, _G)], part_hbm.at[wid, pl.ds(col, _G)])

    tdma = pltpu.async_copy(packed_in.at[pl.ds(0, _N1)], table, tsem)
    zero_bins()
    node_pass(z_in, b_in, _P1 // _NTILES)
    tdma.wait()
    edge_pass(ei_in, _E1 // _NTILES, _E1)
    fold_and_dump(0)

    tdma2 = pltpu.async_copy(packed_out.at[pl.ds(0, _N2)], table.at[pl.ds(0, _N2)], tsem)
    zero_bins()
    node_pass(z_out, b_out, _P2 // _NTILES)
    tdma2.wait()
    edge_pass(ei_out, _E2 // _NTILES, _E2)
    fold_and_dump(_G)


def kernel(x_in, edge_index_in, batch_in, x_out, edge_index_out, batch_out,
           W1a, W1b, W2a, W2b):
    xi = jnp.pad(x_in[:, 0], (0, _P1 - _N1)).reshape(-1, 128)
    bi = jnp.pad(batch_in, (0, _P1 - _N1))
    xo = jnp.pad(x_out[:, 0], (0, _P2 - _N2)).reshape(-1, 128)
    bo = jnp.pad(batch_out, (0, _P2 - _N2))
    w1bt = W1b.reshape(1, 128)
    w2bt = W2b.reshape(1, 128)

    packed_in, z_in = _prep(xi, bi.reshape(-1, 128), W1a, w1bt, W2a, w2bt)
    packed_out, z_out = _prep(xo, bo.reshape(-1, 128), W1a, w1bt, W2a, w2bt)

    part = _sc_main(
        packed_in.reshape(-1), packed_out.reshape(-1),
        edge_index_in.reshape(-1), edge_index_out.reshape(-1),
        z_in.reshape(-1), bi, z_out.reshape(-1), bo,
    )
    total = _reduce(part)[0]
    return (total[:_G].reshape(_G, 1), total[_G:].reshape(_G, 1))


# speedup vs baseline: 551.9554x; 2.1513x over previous
import functools

import jax
import jax.numpy as jnp
from jax import lax
from jax.experimental import pallas as pl
from jax.experimental.pallas import tpu as pltpu
from jax.experimental.pallas import tpu_sc as plsc

_G = 1024
_N1, _E1 = 100000, 6400000
_N2, _E2 = 50000, 1600000
_P1 = 100352
_P2 = 50176
_NTILES = 32
_ECHUNK = 512
_NBUF = 4


def _prep_body(x_ref, b_ref, w1a_ref, w1bt_ref, w2a_ref, w2bt_ref,
               packed_ref, z_ref):
    a1 = w1a_ref[...]
    p1 = a1 * w1bt_ref[...]
    cp1 = jnp.sum(jnp.where(a1 > 0, p1, 0.0))
    cn1 = jnp.sum(jnp.where(a1 < 0, p1, 0.0))
    a2 = w2a_ref[...]
    p2 = a2 * w2bt_ref[...]
    cp2 = jnp.sum(jnp.where(a2 > 0, p2, 0.0))
    cn2 = jnp.sum(jnp.where(a2 < 0, p2, 0.0))
    x = x_ref[...]
    xp = jnp.maximum(x, 0.0)
    xn = jnp.minimum(x, 0.0)
    y = cp1 * xp + cn1 * xn
    z_ref[...] = cp2 * xp + cn2 * xn
    u = lax.bitcast_convert_type(y, jnp.int32)
    u = (u + 0x800) & jnp.int32(-4096)
    packed_ref[...] = u | b_ref[...]


def _prep(x2d, b2d, w1a, w1bt, w2a, w2bt):
    rows = x2d.shape[0]
    return pl.pallas_call(
        _prep_body,
        out_shape=(
            jax.ShapeDtypeStruct((rows, 128), jnp.int32),
            jax.ShapeDtypeStruct((rows, 128), jnp.float32),
        ),
    )(x2d, b2d, w1a, w1bt, w2a, w2bt)


def _reduce_body(part_ref, out_ref):
    out_ref[...] = jnp.sum(part_ref[...], axis=0, keepdims=True)


def _reduce(part):
    return pl.pallas_call(
        _reduce_body,
        out_shape=jax.ShapeDtypeStruct((1, 2 * _G), jnp.float32),
    )(part)


@functools.partial(
    pl.kernel,
    out_type=jax.ShapeDtypeStruct((_NTILES, 2 * _G), jnp.float32),
    mesh=plsc.VectorSubcoreMesh(core_axis_name="c", subcore_axis_name="s"),
    compiler_params=pltpu.CompilerParams(needs_layout_passes=False),
    scratch_types=[
        pltpu.VMEM((_N1,), jnp.int32),
        pltpu.VMEM((2, _ECHUNK), jnp.int32),
        pltpu.VMEM((2, _ECHUNK), jnp.int32),
        pltpu.VMEM((2, _ECHUNK), jnp.int32),
        pltpu.VMEM((2, _ECHUNK), jnp.int32),
        pltpu.VMEM((_P1 // _NTILES,), jnp.float32),
        pltpu.VMEM((_P1 // _NTILES,), jnp.int32),
        pltpu.VMEM((16 * _G,), jnp.float32),
        pltpu.SemaphoreType.DMA,
        pltpu.SemaphoreType.DMA,
        pltpu.SemaphoreType.DMA,
        pltpu.SemaphoreType.DMA,
        pltpu.SemaphoreType.DMA,
    ],
)
def _sc_main(packed_in, packed_out, ei_in, ei_out, z_in, b_in, z_out, b_out,
             part_hbm, table, ebuf0, ebuf1, ebuf2, ebuf3, zbuf, bbuf, bins,
             tsem, esem0, esem1, esem2, esem3):
    wid = lax.axis_index("s") * 2 + lax.axis_index("c")
    hi_mask = jnp.full((16,), -4096, jnp.int32)
    lo_mask = jnp.full((16,), 4095, jnp.int32)
    lane_off = lax.iota(jnp.int32, 16) * _G

    def zero_bins():
        def body(i, _):
            bins[pl.ds(i * 16, 16)] = jnp.zeros((16,), jnp.float32)
            return 0

        lax.fori_loop(0, (16 * _G) // 16, body, 0)

    def node_pass(z_hbm, b_hbm, nn):
        pltpu.sync_copy(z_hbm.at[pl.ds(wid * nn, nn)], zbuf.at[pl.ds(0, nn)])
        pltpu.sync_copy(b_hbm.at[pl.ds(wid * nn, nn)], bbuf.at[pl.ds(0, nn)])

        def body(j, _):
            zv = zbuf[pl.ds(j * 16, 16)]
            bv = bbuf[pl.ds(j * 16, 16)] + lane_off
            plsc.addupdate_scatter(bins, [bv], zv)
            return 0

        lax.fori_loop(0, nn // 16, body, 0)

    ebuf = (ebuf0, ebuf1, ebuf2, ebuf3)
    esem = (esem0, esem1, esem2, esem3)

    def edge_pass(ei_hbm, nedges):
        total_chunks = nedges // _ECHUNK
        ntc = (total_chunks - wid + _NTILES - 1) // _NTILES

        def issue(i, slot):
            col = (i * _NTILES + wid) * _ECHUNK
            pltpu.async_copy(
                ei_hbm.at[:, pl.ds(col, _ECHUNK)], ebuf[slot], esem[slot])

        def drain(slot):
            pltpu.make_async_copy(
                ei_hbm.at[:, pl.ds(0, _ECHUNK)], ebuf[slot], esem[slot]).wait()

        def compute(slot):
            buf = ebuf[slot]

            @plsc.parallel_loop(0, _ECHUNK // 16, unroll=8)
            def body(j):
                vs = buf[0, pl.ds(j * 16, 16)]
                vd = buf[1, pl.ds(j * 16, 16)]
                gs = plsc.load_gather(table, [vs])
                gd = plsc.load_gather(table, [vd])
                y = plsc.bitcast(gs & hi_mask, jnp.float32)
                bd = (gd & lo_mask) + lane_off
                plsc.addupdate_scatter(bins, [bd], y)

        for s in range(_NBUF):
            issue(s, s)

        def group(g, _):
            for s in range(_NBUF):
                k = g * _NBUF + s

                @pl.when(k < ntc)
                def _():
                    drain(s)
                    compute(s)

                    @pl.when(k + _NBUF < ntc)
                    def _():
                        issue(k + _NBUF, s)

            return 0

        lax.fori_loop(0, (ntc + _NBUF - 1) // _NBUF, group, 0)

    def fold_and_dump(col):
        def fold(j, _):
            v = bins[pl.ds(j * 16, 16)]
            for l in range(1, 16):
                v = v + bins[pl.ds(l * _G + j * 16, 16)]
            bins[pl.ds(j * 16, 16)] = v
            return 0

        lax.fori_loop(0, _G // 16, fold, 0)
        pltpu.sync_copy(bins.at[pl.ds(0, _G)], part_hbm.at[wid, pl.ds(col, _G)])

    tdma = pltpu.async_copy(packed_in.at[pl.ds(0, _N1)], table, tsem)
    zero_bins()
    node_pass(z_in, b_in, _P1 // _NTILES)
    tdma.wait()
    edge_pass(ei_in, _E1)
    tdma2 = pltpu.async_copy(packed_out.at[pl.ds(0, _N2)], table.at[pl.ds(0, _N2)], tsem)
    fold_and_dump(0)

    zero_bins()
    node_pass(z_out, b_out, _P2 // _NTILES)
    tdma2.wait()
    edge_pass(ei_out, _E2)
    fold_and_dump(_G)


def kernel(x_in, edge_index_in, batch_in, x_out, edge_index_out, batch_out,
           W1a, W1b, W2a, W2b):
    xi = jnp.pad(x_in[:, 0], (0, _P1 - _N1)).reshape(-1, 128)
    bi = jnp.pad(batch_in, (0, _P1 - _N1))
    xo = jnp.pad(x_out[:, 0], (0, _P2 - _N2)).reshape(-1, 128)
    bo = jnp.pad(batch_out, (0, _P2 - _N2))
    w1bt = W1b.reshape(1, 128)
    w2bt = W2b.reshape(1, 128)

    packed_in, z_in = _prep(xi, bi.reshape(-1, 128), W1a, w1bt, W2a, w2bt)
    packed_out, z_out = _prep(xo, bo.reshape(-1, 128), W1a, w1bt, W2a, w2bt)

    part = _sc_main(
        packed_in.reshape(-1), packed_out.reshape(-1),
        edge_index_in, edge_index_out,
        z_in.reshape(-1), bi, z_out.reshape(-1), bo,
    )
    total = _reduce(part)[0]
    return (total[:_G].reshape(_G, 1), total[_G:].reshape(_G, 1))
